# baseline (speedup 1.0000x reference)
import jax
import jax.numpy as jnp
from jax.experimental import pallas as pl
from jax.experimental.pallas import tpu as pltpu

TOKENS = 16384
IN_FEATURES = 2048
NUM_EXPERTS = 64
BT = 2048
NCHUNK = TOKENS // BT
NBUF = 2


def _copy_in(x_hbm, xbuf, sem, chunk, slot):
    return pltpu.make_async_copy(
        x_hbm.at[pl.ds(chunk * BT, BT), :], xbuf.at[slot], sem.at[slot])


def _router_kernel(x_hbm, w_ref, b_ref, gates_ref, idx_ref, xbuf, sem):
    w = w_ref[...]
    b = b_ref[...]

    for s in range(NBUF):
        _copy_in(x_hbm, xbuf, sem, s, s).start()

    def step(i, carry):
        slot = jax.lax.rem(i, NBUF)
        _copy_in(x_hbm, xbuf, sem, i, slot).wait()
        x = xbuf[slot]
        logits = jax.lax.dot_general(
            x, w, (((1,), (1,)), ((), ())),
            preferred_element_type=jnp.float32) + b
        m = jnp.max(logits, axis=-1, keepdims=True)
        e = jnp.exp(logits - m)
        s = jnp.sum(e, axis=-1, keepdims=True)

        fiota = jax.lax.broadcasted_iota(jnp.int32, logits.shape, 1).astype(
            jnp.float32)
        i1 = jnp.min(jnp.where(logits == m, fiota, float(NUM_EXPERTS)),
                     axis=-1, keepdims=True)
        masked = jnp.where(fiota == i1, -jnp.inf, logits)
        v2 = jnp.max(masked, axis=-1, keepdims=True)
        i2 = jnp.min(jnp.where(masked == v2, fiota, float(NUM_EXPERTS)),
                     axis=-1, keepdims=True)
        g1 = 1.0 / s
        g2 = jnp.exp(v2 - m) / s

        gates_ref[pl.ds(i * BT, BT), :] = jnp.concatenate([g1, g2], axis=-1)
        idx_ref[pl.ds(i * BT, BT), :] = jnp.concatenate(
            [i1, i2], axis=-1).astype(jnp.int32)

        @pl.when(i + NBUF < NCHUNK)
        def _():
            _copy_in(x_hbm, xbuf, sem, i + NBUF, slot).start()

        return carry

    jax.lax.fori_loop(0, NCHUNK, step, 0)


def kernel(x, W, b):
    b2 = b.reshape(1, NUM_EXPERTS)
    gates, idx = pl.pallas_call(
        _router_kernel,
        in_specs=[
            pl.BlockSpec(memory_space=pltpu.HBM),
            pl.BlockSpec(memory_space=pltpu.VMEM),
            pl.BlockSpec(memory_space=pltpu.VMEM),
        ],
        out_specs=[
            pl.BlockSpec(memory_space=pltpu.VMEM),
            pl.BlockSpec(memory_space=pltpu.VMEM),
        ],
        out_shape=[
            jax.ShapeDtypeStruct((TOKENS, 2), jnp.float32),
            jax.ShapeDtypeStruct((TOKENS, 2), jnp.int32),
        ],
        scratch_shapes=[
            pltpu.VMEM((NBUF, BT, IN_FEATURES), jnp.float32),
            pltpu.SemaphoreType.DMA((NBUF,)),
        ],
    )(x, W, b2)
    return (gates, idx)


# speedup vs baseline: 1.0436x; 1.0436x over previous
import jax
import jax.numpy as jnp
from jax.experimental import pallas as pl

TOKENS = 16384
IN_FEATURES = 2048
NUM_EXPERTS = 64
BT = 2048


def _router_kernel(x_ref, w_ref, b_ref, gates_ref, idx_ref):
    x = x_ref[...]
    w = w_ref[...]
    logits = jax.lax.dot_general(
        x, w, (((1,), (1,)), ((), ())),
        preferred_element_type=jnp.float32) + b_ref[...]
    m = jnp.max(logits, axis=-1, keepdims=True)
    e = jnp.exp(logits - m)
    s = jnp.sum(e, axis=-1, keepdims=True)

    fiota = jax.lax.broadcasted_iota(jnp.int32, logits.shape, 1).astype(
        jnp.float32)
    i1 = jnp.min(jnp.where(logits == m, fiota, float(NUM_EXPERTS)),
                 axis=-1, keepdims=True)
    masked = jnp.where(fiota == i1, -jnp.inf, logits)
    v2 = jnp.max(masked, axis=-1, keepdims=True)
    i2 = jnp.min(jnp.where(masked == v2, fiota, float(NUM_EXPERTS)),
                 axis=-1, keepdims=True)
    g1 = 1.0 / s
    g2 = jnp.exp(v2 - m) / s

    gates_ref[...] = jnp.concatenate([g1, g2], axis=-1)
    idx_ref[...] = jnp.concatenate([i1, i2], axis=-1).astype(jnp.int32)


def kernel(x, W, b):
    b2 = b.reshape(1, NUM_EXPERTS)
    grid = (TOKENS // BT,)
    gates, idx = pl.pallas_call(
        _router_kernel,
        grid=grid,
        in_specs=[
            pl.BlockSpec((BT, IN_FEATURES), lambda i: (i, 0)),
            pl.BlockSpec((NUM_EXPERTS, IN_FEATURES), lambda i: (0, 0)),
            pl.BlockSpec((1, NUM_EXPERTS), lambda i: (0, 0)),
        ],
        out_specs=[
            pl.BlockSpec((BT, 2), lambda i: (i, 0)),
            pl.BlockSpec((BT, 2), lambda i: (i, 0)),
        ],
        out_shape=[
            jax.ShapeDtypeStruct((TOKENS, 2), jnp.float32),
            jax.ShapeDtypeStruct((TOKENS, 2), jnp.int32),
        ],
    )(x, W, b2)
    return (gates, idx)
